# Initial kernel scaffold; baseline (speedup 1.0000x reference)
import jax
import jax.numpy as jnp
from jax.experimental import pallas as pl


def kernel(input, rand_noise, replace_noise):
    raise NotImplementedError("write your pallas kernel here")


# speedup vs baseline: 5.4361x; 5.4361x over previous
import functools

import jax
import jax.numpy as jnp
from jax.experimental import pallas as pl

_BATCH, _SEQ = 4, 8192
_MAX_MASKED = 1229


def _body(inp_ref, rand_ref, rep_ref, out_masked_ref, out_labels_ref):
    inp = inp_ref[...]
    rand = rand_ref[...]
    cand = jnp.logical_not((inp == 0) | (inp == 101) | (inp == 102))
    candf = jnp.where(cand, jnp.float32(1.0), jnp.float32(0.0))
    idx = jax.lax.broadcasted_iota(jnp.int32, (_BATCH, _SEQ), 1)

    num_tokens = jnp.sum(candf, axis=1, keepdims=True)
    kq = jnp.ceil(num_tokens * jnp.float32(0.15))

    def t_step(_, carry):
        lo, hi = carry
        mid = lo + (hi - lo) // 2
        c = jnp.sum(jnp.where(cand & (idx <= mid), jnp.float32(1.0),
                              jnp.float32(0.0)), axis=1, keepdims=True)
        pred = c > kq
        return (jnp.where(pred, lo, mid), jnp.where(pred, mid, hi))

    lo0 = jnp.full((_BATCH, 1), -1, jnp.int32)
    hi0 = jnp.full((_BATCH, 1), _SEQ - 1, jnp.int32)
    lo, hi = jax.lax.fori_loop(0, 13, t_step, (lo0, hi0))
    c_hi = jnp.sum(jnp.where(cand & (idx <= hi), jnp.float32(1.0),
                             jnp.float32(0.0)), axis=1, keepdims=True)
    p = jnp.where(c_hi > kq, hi, jnp.int32(_SEQ))
    t_b = jnp.minimum(p, jnp.int32(_MAX_MASKED))

    bits = jax.lax.bitcast_convert_type(rand, jnp.int32)
    u = jnp.where(cand, bits + jnp.int32(1 << 30), jnp.int32(0))

    def v_step(_, carry):
        lo, hi = carry
        mid = lo + (hi - lo) // 2
        c = jnp.sum(jnp.where(u >= mid, jnp.int32(1), jnp.int32(0)),
                    axis=1, keepdims=True)
        pred = c >= t_b
        return (jnp.where(pred, mid, lo), jnp.where(pred, hi, mid))

    vlo0 = jnp.zeros((_BATCH, 1), jnp.int32)
    vhi0 = jnp.full((_BATCH, 1), 0x7FFFFFFF, jnp.int32)
    vlo, _ = jax.lax.fori_loop(0, 31, v_step, (vlo0, vhi0))
    v_star = vlo

    n_gt = jnp.sum(jnp.where(u > v_star, jnp.int32(1), jnp.int32(0)),
                   axis=1, keepdims=True)
    n_tie = t_b - n_gt
    match = u == v_star

    def c_step(_, carry):
        lo, hi = carry
        mid = lo + (hi - lo) // 2
        c = jnp.sum(jnp.where(match & (idx <= mid), jnp.int32(1),
                              jnp.int32(0)), axis=1, keepdims=True)
        pred = c >= n_tie
        return (jnp.where(pred, lo, mid), jnp.where(pred, mid, hi))

    clo0 = jnp.full((_BATCH, 1), -1, jnp.int32)
    chi0 = jnp.full((_BATCH, 1), _SEQ - 1, jnp.int32)
    _, chi = jax.lax.fori_loop(0, 13, c_step, (clo0, chi0))
    c_star = chi

    sel = (u > v_star) | (match & (idx <= c_star))
    rep = rep_ref[...] < jnp.float32(0.9)
    out_masked_ref[...] = jnp.where(sel & rep, jnp.int32(103), inp)
    out_labels_ref[...] = jnp.where(sel, inp, jnp.int32(0))


@functools.partial(jax.jit, static_argnames=("interpret",))
def kernel(input, rand_noise, replace_noise, interpret=False):
    out = pl.pallas_call(
        _body,
        out_shape=(
            jax.ShapeDtypeStruct((_BATCH, _SEQ), jnp.int32),
            jax.ShapeDtypeStruct((_BATCH, _SEQ), jnp.int32),
        ),
        interpret=interpret,
    )(input, rand_noise, replace_noise)
    return out
